# baseline (speedup 1.0000x reference)
import functools

import jax
import jax.numpy as jnp
from jax.experimental import pallas as pl
from jax.experimental.pallas import tpu as pltpu


def _tdot(lhs, rhs):
    return jax.lax.dot_general(lhs, rhs, (((0,), (0,)), ((), ())),
                               preferred_element_type=jnp.float32)


def _body(A_ref, X_ref, W1a_ref, W1b_ref, b1_ref, Wp_ref, bp_ref,
          W2a_ref, W2b_ref, b2_ref, Wd_ref, bd_ref,
          out_ref, P_ref, XWb_ref, Avm_ref, S_ref, h_ref, rA_ref, dg_ref,
          *, BN, NB, K, H):
    k = pl.program_id(0)

    @pl.when(k == 0)
    def _init():
        P_ref[...] = jnp.zeros_like(P_ref)
        P_ref[:, 0:H] = jnp.dot(X_ref[...], W1a_ref[...],
                                preferred_element_type=jnp.float32)
        P_ref[:, H:H + 1] = jnp.ones_like(P_ref[:, H:H + 1])
        XWb_ref[...] = jnp.dot(X_ref[...], W1b_ref[...],
                               preferred_element_type=jnp.float32) + b1_ref[...]
        dg_ref[...] = jnp.zeros_like(dg_ref)

    A_b = A_ref[...]
    Ab_bf = A_b.astype(jnp.bfloat16)
    Avm_ref[pl.ds(k * BN, BN), :] = Ab_bf
    hpre = jnp.dot(A_b, P_ref[...], preferred_element_type=jnp.float32)
    rA_ref[pl.ds(k * BN, BN), :] = hpre[:, H:H + 1]
    h = jnp.maximum(hpre[:, 0:H] + XWb_ref[pl.ds(k * BN, BN), :], 0.0)
    h_ref[pl.ds(k * BN, BN), :] = h.astype(jnp.bfloat16)
    logits = jnp.dot(h, Wp_ref[...],
                     preferred_element_type=jnp.float32) + bp_ref[...]
    m = jnp.max(logits, axis=-1, keepdims=True)
    e = jnp.exp(logits - m)
    S_b = (e * (1.0 / jnp.sum(e, axis=-1, keepdims=True))).astype(jnp.bfloat16)
    S_ref[pl.ds(k * BN, BN), :] = S_b
    S_b32 = S_b.astype(jnp.float32)

    def _new_row(c, _):
        t = jnp.dot(Avm_ref[pl.ds(k * BN, BN), pl.ds(c * BN, BN)],
                    S_ref[pl.ds(c * BN, BN), :],
                    preferred_element_type=jnp.float32)
        dg_ref[...] += jnp.sum(S_b32 * t, axis=0, keepdims=True)
        return 0

    jax.lax.fori_loop(0, k + 1, _new_row, 0)

    def _old_rows(r, _):
        t = jnp.dot(Avm_ref[pl.ds(r * BN, BN), pl.ds(k * BN, BN)], S_b,
                    preferred_element_type=jnp.float32)
        Sr = S_ref[pl.ds(r * BN, BN), :].astype(jnp.float32)
        dg_ref[...] += jnp.sum(Sr * t, axis=0, keepdims=True)
        return 0

    jax.lax.fori_loop(0, k, _old_rows, 0)

    @pl.when(k == NB - 1)
    def _final():
        S = S_ref[...]
        N = S.shape[0]
        xp = _tdot(S, h_ref[...])
        dg_row = dg_ref[...]
        rr = jax.lax.broadcasted_iota(jnp.int32, (K, K), 0)
        cc = jax.lax.broadcasted_iota(jnp.int32, (K, K), 1)
        dgb = jnp.broadcast_to(dg_row, (K, K))
        diag_col = jnp.sum(jnp.where(rr == cc, dgb, 0.0), axis=1,
                           keepdims=True)
        rAc = (rA_ref[...] - (0.5 * N)).astype(jnp.bfloat16)
        ones_col = jnp.ones((N, 1), dtype=jnp.bfloat16)
        d = (0.5 * N) * _tdot(S, ones_col) + _tdot(S, rAc) - diag_col
        dinv = jax.lax.rsqrt(d + 1e-9)
        u = jnp.dot(xp, W2a_ref[...], preferred_element_type=jnp.float32)
        w = u * dinv
        Sw = jnp.dot(S, w.astype(jnp.bfloat16),
                     preferred_element_type=jnp.float32)
        ASw = jnp.dot(Avm_ref[...], Sw.astype(jnp.bfloat16),
                      preferred_element_type=jnp.float32)
        y = _tdot(S, ASw.astype(jnp.bfloat16))
        v = (y - diag_col * w) * dinv
        h2 = v + jnp.dot(xp, W2b_ref[...],
                         preferred_element_type=jnp.float32) + b2_ref[...]
        h2 = jnp.maximum(h2, 0.0)
        out_ref[...] = jnp.dot(h2, Wd_ref[...],
                               preferred_element_type=jnp.float32) + bd_ref[...]


def kernel(x, a, i, W1a, W1b, b1, Wp, bp, W2a, W2b, b2, Wd, bd):
    N, F = x.shape
    H = W1a.shape[1]
    K = Wp.shape[1]
    BN = 512
    NB = N // BN
    body = functools.partial(_body, BN=BN, NB=NB, K=K, H=H)
    full = lambda b: (0, 0)
    out = pl.pallas_call(
        body,
        grid=(NB,),
        in_specs=[
            pl.BlockSpec((BN, N), lambda b: (b, 0)),
            pl.BlockSpec((N, F), full),
            pl.BlockSpec((F, H), full),
            pl.BlockSpec((F, H), full),
            pl.BlockSpec((1, H), full),
            pl.BlockSpec((H, K), full),
            pl.BlockSpec((1, K), full),
            pl.BlockSpec((H, H), full),
            pl.BlockSpec((H, H), full),
            pl.BlockSpec((1, H), full),
            pl.BlockSpec((H, 1), full),
            pl.BlockSpec((1, 1), full),
        ],
        out_specs=pl.BlockSpec((K, 1), full),
        out_shape=jax.ShapeDtypeStruct((K, 1), jnp.float32),
        scratch_shapes=[
            pltpu.VMEM((N, 2 * H), jnp.float32),
            pltpu.VMEM((N, H), jnp.float32),
            pltpu.VMEM((N, N), jnp.bfloat16),
            pltpu.VMEM((N, K), jnp.bfloat16),
            pltpu.VMEM((N, H), jnp.bfloat16),
            pltpu.VMEM((N, 1), jnp.float32),
            pltpu.VMEM((1, K), jnp.float32),
        ],
    )(a, x, W1a, W1b, b1.reshape(1, H), Wp, bp.reshape(1, K),
      W2a, W2b, b2.reshape(1, H), Wd, bd.reshape(1, 1))
    return out


# speedup vs baseline: 1.2342x; 1.2342x over previous
import functools

import jax
import jax.numpy as jnp
from jax.experimental import pallas as pl
from jax.experimental.pallas import tpu as pltpu


def _body(A_ref, X_ref, W1a_ref, W1b_ref, b1_ref, Wp_ref, bp_ref,
          W2a_ref, W2b_ref, b2_ref, Wd_ref, bd_ref,
          out_ref, P_ref, Avm_ref, S_ref, h_ref, AS_ref, *, BN, NB, K):
    p = pl.program_id(0)
    b = pl.program_id(1)

    @pl.when(jnp.logical_and(p == 0, b == 0))
    def _init():
        P_ref[...] = jnp.dot(X_ref[...], W1a_ref[...],
                             preferred_element_type=jnp.float32)

    @pl.when(p == 0)
    def _phase0():
        A_b = A_ref[...]
        Avm_ref[pl.ds(b * BN, BN), :] = A_b.astype(jnp.bfloat16)
        X_b = X_ref[pl.ds(b * BN, BN), :]
        h = jnp.dot(A_b, P_ref[...], preferred_element_type=jnp.float32)
        h = h + jnp.dot(X_b, W1b_ref[...],
                        preferred_element_type=jnp.float32) + b1_ref[...]
        h = jnp.maximum(h, 0.0)
        h_ref[pl.ds(b * BN, BN), :] = h
        logits = jnp.dot(h, Wp_ref[...],
                         preferred_element_type=jnp.float32) + bp_ref[...]
        m = jnp.max(logits, axis=-1, keepdims=True)
        e = jnp.exp(logits - m)
        S_b = e * (1.0 / jnp.sum(e, axis=-1, keepdims=True))
        S_ref[pl.ds(b * BN, BN), :] = S_b.astype(jnp.bfloat16)

    @pl.when(p == 1)
    def _phase1():
        A_b = Avm_ref[pl.ds(b * BN, BN), :]
        AS = jnp.dot(A_b, S_ref[...], preferred_element_type=jnp.float32)
        AS_ref[pl.ds(b * BN, BN), :] = AS.astype(jnp.bfloat16)

    @pl.when(jnp.logical_and(p == 1, b == NB - 1))
    def _final():
        S = S_ref[...]
        xp = jax.lax.dot_general(
            S, h_ref[...].astype(jnp.bfloat16), (((0,), (0,)), ((), ())),
            preferred_element_type=jnp.float32)
        ap = jax.lax.dot_general(
            S, AS_ref[...], (((0,), (0,)), ((), ())),
            preferred_element_type=jnp.float32)
        r = jax.lax.broadcasted_iota(jnp.int32, (K, K), 0)
        c = jax.lax.broadcasted_iota(jnp.int32, (K, K), 1)
        ap = jnp.where(r == c, 0.0, ap)
        d = jnp.sum(ap, axis=1, keepdims=True)
        dinv = jax.lax.rsqrt(d + 1e-9)
        u = jnp.dot(xp, W2a_ref[...], preferred_element_type=jnp.float32)
        v = jnp.dot(ap, u * dinv, preferred_element_type=jnp.float32) * dinv
        h2 = v + jnp.dot(xp, W2b_ref[...],
                         preferred_element_type=jnp.float32) + b2_ref[...]
        h2 = jnp.maximum(h2, 0.0)
        out_ref[...] = jnp.dot(h2, Wd_ref[...],
                               preferred_element_type=jnp.float32) + bd_ref[...]


def kernel(x, a, i, W1a, W1b, b1, Wp, bp, W2a, W2b, b2, Wd, bd):
    N, F = x.shape
    H = W1a.shape[1]
    K = Wp.shape[1]
    BN = 512
    NB = N // BN
    body = functools.partial(_body, BN=BN, NB=NB, K=K)
    full = lambda p, b: (0, 0)
    out = pl.pallas_call(
        body,
        grid=(2, NB),
        in_specs=[
            pl.BlockSpec((BN, N), lambda p, b: ((1 - p) * b + p * (NB - 1), 0)),
            pl.BlockSpec((N, F), full),
            pl.BlockSpec((F, H), full),
            pl.BlockSpec((F, H), full),
            pl.BlockSpec((1, H), full),
            pl.BlockSpec((H, K), full),
            pl.BlockSpec((1, K), full),
            pl.BlockSpec((H, H), full),
            pl.BlockSpec((H, H), full),
            pl.BlockSpec((1, H), full),
            pl.BlockSpec((H, 1), full),
            pl.BlockSpec((1, 1), full),
        ],
        out_specs=pl.BlockSpec((K, 1), full),
        out_shape=jax.ShapeDtypeStruct((K, 1), jnp.float32),
        scratch_shapes=[
            pltpu.VMEM((N, H), jnp.float32),
            pltpu.VMEM((N, N), jnp.bfloat16),
            pltpu.VMEM((N, K), jnp.bfloat16),
            pltpu.VMEM((N, H), jnp.float32),
            pltpu.VMEM((N, K), jnp.bfloat16),
        ],
    )(a, x, W1a, W1b, b1.reshape(1, H), Wp, bp.reshape(1, K),
      W2a, W2b, b2.reshape(1, H), Wd, bd.reshape(1, 1))
    return out


# speedup vs baseline: 1.2504x; 1.0131x over previous
import functools

import jax
import jax.numpy as jnp
from jax.experimental import pallas as pl
from jax.experimental.pallas import tpu as pltpu


def _body(A_ref, X_ref, W1a_ref, W1b_ref, b1_ref, Wp_ref, bp_ref,
          W2a_ref, W2b_ref, b2_ref, Wd_ref, bd_ref,
          out_ref, P_ref, Avm_ref, S_ref, h_ref, AS_ref, *, BN, NB, K):
    p = pl.program_id(0)
    b = pl.program_id(1)

    @pl.when(jnp.logical_and(p == 0, b == 0))
    def _init():
        P_ref[...] = jnp.dot(X_ref[...], W1a_ref[...],
                             preferred_element_type=jnp.float32)

    @pl.when(p == 0)
    def _phase0():
        A_b = A_ref[...]
        Avm_ref[pl.ds(b * BN, BN), :] = A_b.astype(jnp.bfloat16)
        X_b = X_ref[pl.ds(b * BN, BN), :]
        h = jnp.dot(A_b, P_ref[...], preferred_element_type=jnp.float32)
        h = h + jnp.dot(X_b, W1b_ref[...],
                        preferred_element_type=jnp.float32) + b1_ref[...]
        h = jnp.maximum(h, 0.0)
        h_ref[pl.ds(b * BN, BN), :] = h
        logits = jnp.dot(h, Wp_ref[...],
                         preferred_element_type=jnp.float32) + bp_ref[...]
        m = jnp.max(logits, axis=-1, keepdims=True)
        e = jnp.exp(logits - m)
        S_b = e * (1.0 / jnp.sum(e, axis=-1, keepdims=True))
        S_ref[pl.ds(b * BN, BN), :] = S_b.astype(jnp.bfloat16)

    @pl.when(p == 1)
    def _phase1():
        A_b = Avm_ref[pl.ds(b * BN, BN), :]
        AS = jnp.dot(A_b, S_ref[...], preferred_element_type=jnp.float32)
        AS_ref[pl.ds(b * BN, BN), :] = AS.astype(jnp.bfloat16)

    @pl.when(jnp.logical_and(p == 1, b == NB - 1))
    def _final():
        S = S_ref[...]
        xp = jax.lax.dot_general(
            S, h_ref[...].astype(jnp.bfloat16), (((0,), (0,)), ((), ())),
            preferred_element_type=jnp.float32)
        ap = jax.lax.dot_general(
            S, AS_ref[...], (((0,), (0,)), ((), ())),
            preferred_element_type=jnp.float32)
        r = jax.lax.broadcasted_iota(jnp.int32, (K, K), 0)
        c = jax.lax.broadcasted_iota(jnp.int32, (K, K), 1)
        ap = jnp.where(r == c, 0.0, ap)
        d = jnp.sum(ap, axis=1, keepdims=True)
        dinv = jax.lax.rsqrt(d + 1e-9)
        u = jnp.dot(xp, W2a_ref[...], preferred_element_type=jnp.float32)
        v = jnp.dot(ap, u * dinv, preferred_element_type=jnp.float32) * dinv
        h2 = v + jnp.dot(xp, W2b_ref[...],
                         preferred_element_type=jnp.float32) + b2_ref[...]
        h2 = jnp.maximum(h2, 0.0)
        out_ref[...] = jnp.dot(h2, Wd_ref[...],
                               preferred_element_type=jnp.float32) + bd_ref[...]


def kernel(x, a, i, W1a, W1b, b1, Wp, bp, W2a, W2b, b2, Wd, bd):
    N, F = x.shape
    H = W1a.shape[1]
    K = Wp.shape[1]
    BN = 1024
    NB = N // BN
    body = functools.partial(_body, BN=BN, NB=NB, K=K)
    full = lambda p, b: (0, 0)
    out = pl.pallas_call(
        body,
        grid=(2, NB),
        in_specs=[
            pl.BlockSpec((BN, N), lambda p, b: ((1 - p) * b + p * (NB - 1), 0)),
            pl.BlockSpec((N, F), full),
            pl.BlockSpec((F, H), full),
            pl.BlockSpec((F, H), full),
            pl.BlockSpec((1, H), full),
            pl.BlockSpec((H, K), full),
            pl.BlockSpec((1, K), full),
            pl.BlockSpec((H, H), full),
            pl.BlockSpec((H, H), full),
            pl.BlockSpec((1, H), full),
            pl.BlockSpec((H, 1), full),
            pl.BlockSpec((1, 1), full),
        ],
        out_specs=pl.BlockSpec((K, 1), full),
        out_shape=jax.ShapeDtypeStruct((K, 1), jnp.float32),
        scratch_shapes=[
            pltpu.VMEM((N, H), jnp.float32),
            pltpu.VMEM((N, N), jnp.bfloat16),
            pltpu.VMEM((N, K), jnp.bfloat16),
            pltpu.VMEM((N, H), jnp.float32),
            pltpu.VMEM((N, K), jnp.bfloat16),
        ],
    )(a, x, W1a, W1b, b1.reshape(1, H), Wp, bp.reshape(1, K),
      W2a, W2b, b2.reshape(1, H), Wd, bd.reshape(1, 1))
    return out
